# Initial kernel scaffold; baseline (speedup 1.0000x reference)
import jax
import jax.numpy as jnp
from jax.experimental import pallas as pl


def kernel(x_ori, gamora0, gamora1, gamora2, edge_index, Wl, bl, Wr, bn_g, bn_b, m1_W1, m1_b1, m1_W2, m1_b2, bn2_g, bn2_b, m2_W1, m2_b1, m2_W2, m2_b2):
    raise NotImplementedError("write your pallas kernel here")


# speedup vs baseline: 2.6389x; 2.6389x over previous
import functools

import jax
import jax.numpy as jnp
from jax import lax
from jax.experimental import pallas as pl
from jax.experimental.pallas import tpu as pltpu
from jax.experimental.pallas import tpu_sc as plsc

_N = 10000
_E = 320000
_H = 128
_OUT = 2
_MNN = 1000
_L = 4
_EPS = 1e-5

_C = 128
_EPAD = 327680
_NCHUNK = _EPAD // _C
_NS = 16
_NC = 2
_CPT = _NCHUNK // (_NC * _NS)
_RPT = 632
_NROW = _NS * _RPT
_DW = 128


_MESH = plsc.VectorSubcoreMesh(core_axis_name="c", subcore_axis_name="s")


def _sc_agg_body(x_hbm, src_hbm, dst_hbm, zeros_hbm, agg_hbm,
                 src_v, dst_v, rows_v, acc_sh, gsem):
    cid = lax.axis_index("c")
    sid = lax.axis_index("s")
    rows = pl.ds(sid * _RPT, _RPT)
    chunks = pl.ds((cid * _NS + sid) * _CPT, _CPT)

    pltpu.sync_copy(src_hbm.at[chunks], src_v)
    pltpu.sync_copy(dst_hbm.at[chunks], dst_v)
    pltpu.sync_copy(zeros_hbm, acc_sh.at[rows])
    plsc.subcore_barrier()

    def chunk(k, carry):
        pltpu.async_copy(x_hbm.at[src_v.at[k]], rows_v, gsem).wait()
        pltpu.sync_copy(rows_v, acc_sh.at[dst_v.at[k]], add=True)
        return carry

    lax.fori_loop(0, _CPT, chunk, 0, unroll=False)
    plsc.subcore_barrier()

    pltpu.sync_copy(acc_sh.at[rows], agg_hbm.at[cid, rows])


_sc_agg = pl.kernel(
    _sc_agg_body,
    out_type=jax.ShapeDtypeStruct((_NC, _NROW, _H), jnp.float32),
    mesh=_MESH,
    scratch_types=(
        pltpu.VMEM((_CPT, _C), jnp.int32),
        pltpu.VMEM((_CPT, _C), jnp.int32),
        pltpu.VMEM((_C, _H), jnp.float32),
        pltpu.VMEM_SHARED((_NROW, _H), jnp.float32),
        pltpu.SemaphoreType.DMA,
    ),
)


def _sc_deg_body(dst_hbm, zerosd_hbm, ones_hbm, deg_hbm,
                 dst_v, ones_v, deg_sh, gsem):
    cid = lax.axis_index("c")
    sid = lax.axis_index("s")
    rows = pl.ds(sid * _RPT, _RPT)
    chunks = pl.ds((cid * _NS + sid) * _CPT, _CPT)

    pltpu.sync_copy(dst_hbm.at[chunks], dst_v)
    pltpu.sync_copy(zerosd_hbm, deg_sh.at[rows])
    pltpu.sync_copy(ones_hbm, ones_v)
    plsc.subcore_barrier()

    def chunk(k, carry):
        pltpu.sync_copy(ones_v, deg_sh.at[dst_v.at[k]], add=True)
        return carry

    lax.fori_loop(0, _CPT, chunk, 0, unroll=False)
    plsc.subcore_barrier()
    pltpu.sync_copy(deg_sh.at[rows], deg_hbm.at[cid, rows])


_sc_deg = pl.kernel(
    _sc_deg_body,
    out_type=jax.ShapeDtypeStruct((_NC, _NROW, _DW), jnp.float32),
    mesh=_MESH,
    scratch_types=(
        pltpu.VMEM((_CPT, _C), jnp.int32),
        pltpu.VMEM((_C, _DW), jnp.float32),
        pltpu.VMEM_SHARED((_NROW, _DW), jnp.float32),
        pltpu.SemaphoreType.DMA,
    ),
)


def _tc_layer_body(aggp, x_ref, degw, wl, wr, b, g, bb, out):
    agg = aggp[0, :_N, :] + aggp[1, :_N, :]
    x = x_ref[...]
    deg = jnp.maximum(degw[0, :_N, 0:1] + degw[1, :_N, 0:1], 1.0)
    y = (jnp.dot(agg / deg, wl[...], preferred_element_type=jnp.float32)
         + jnp.dot(x, wr[...], preferred_element_type=jnp.float32)
         + b[...])
    mu = jnp.mean(y, axis=0, keepdims=True)
    var = jnp.mean((y - mu) ** 2, axis=0, keepdims=True)
    y = (y - mu) * lax.rsqrt(var + _EPS) * g[...] + bb[...]
    out[...] = jnp.maximum(y, 0.0)


_tc_layer = pl.pallas_call(
    _tc_layer_body,
    out_shape=jax.ShapeDtypeStruct((_N, _H), jnp.float32),
)


def _tc_head1_body(x2d, w1, b1, w2, b2, out):
    h = jnp.dot(x2d[...], w1[...], preferred_element_type=jnp.float32)
    h = jnp.maximum(h + b1[...], 0.0)
    out[...] = jnp.dot(h, w2[...], preferred_element_type=jnp.float32) + b2[...]


_tc_head1 = pl.pallas_call(
    _tc_head1_body,
    out_shape=jax.ShapeDtypeStruct((_N * _H // _MNN, 1), jnp.float32),
)


def _tc_head2_body(t, g, bb, w1, b1, w2, b2, out):
    x = t[...]
    mu = jnp.mean(x, axis=0, keepdims=True)
    var = jnp.mean((x - mu) ** 2, axis=0, keepdims=True)
    x = (x - mu) * lax.rsqrt(var + _EPS) * g[...] + bb[...]
    x = jnp.maximum(x, 0.0)
    h = jnp.maximum(
        jnp.dot(x, w1[...], preferred_element_type=jnp.float32) + b1[...], 0.0)
    out[...] = (jnp.dot(h, w2[...], preferred_element_type=jnp.float32)
                + b2[...])


_tc_head2 = pl.pallas_call(
    _tc_head2_body,
    out_shape=jax.ShapeDtypeStruct((_N * _H // _MNN // _H, _OUT), jnp.float32),
)


def kernel(x_ori, gamora0, gamora1, gamora2, edge_index, Wl, bl, Wr, bn_g,
           bn_b, m1_W1, m1_b1, m1_W2, m1_b2, bn2_g, bn2_b, m2_W1, m2_b1,
           m2_W2, m2_b2):
    x = jnp.concatenate([x_ori, gamora0, gamora1, gamora2], axis=1)

    pad = _EPAD - _E
    src = jnp.concatenate(
        [edge_index[0], jnp.zeros((pad,), jnp.int32)]).reshape(_NCHUNK, _C)
    dst = jnp.concatenate(
        [edge_index[1], jnp.full((pad,), _N, jnp.int32)]).reshape(_NCHUNK, _C)

    zeros = jnp.zeros((_RPT, _H), jnp.float32)
    zerosd = jnp.zeros((_RPT, _DW), jnp.float32)
    ones = jnp.ones((_C, _DW), jnp.float32)

    degw = _sc_deg(dst, zerosd, ones)
    for i in range(_L):
        aggp = _sc_agg(x, src, dst, zeros)
        x = _tc_layer(aggp, x, degw, Wl[i], Wr[i], bl[i].reshape(1, _H),
                      bn_g[i].reshape(1, _H), bn_b[i].reshape(1, _H))

    x2d = x.reshape(_N * _H // _MNN, _MNN)
    t = _tc_head1(x2d, m1_W1, m1_b1.reshape(1, _H), m1_W2,
                  m1_b2.reshape(1, 1))
    t10 = t.reshape(_N * _H // _MNN // _H, _H)
    out = _tc_head2(t10, bn2_g.reshape(1, _H), bn2_b.reshape(1, _H),
                    m2_W1, m2_b1.reshape(1, _H), m2_W2,
                    m2_b2.reshape(1, _OUT))
    return out


# speedup vs baseline: 3.0885x; 1.1704x over previous
import functools

import jax
import jax.numpy as jnp
from jax import lax
from jax.experimental import pallas as pl
from jax.experimental.pallas import tpu as pltpu
from jax.experimental.pallas import tpu_sc as plsc

_N = 10000
_E = 320000
_H = 128
_OUT = 2
_MNN = 1000
_L = 4
_EPS = 1e-5

_C = 128
_EPAD = 327680
_NCHUNK = _EPAD // _C
_NS = 16
_NC = 2
_CPT = _NCHUNK // (_NC * _NS)
_RPT = 632
_NROW = _NS * _RPT
_DW = 128


_MESH = plsc.VectorSubcoreMesh(core_axis_name="c", subcore_axis_name="s")


def _sc_agg_body(x_hbm, packed_hbm, agg_hbm,
                 packed_v, srcr, dstr, rows0, rows1, acc_sh, sem0, sem1):
    cid = lax.axis_index("c")
    sid = lax.axis_index("s")
    rows = pl.ds(sid * _RPT, _RPT)
    chunks = pl.ds((cid * _NS + sid) * _CPT, _CPT)

    pltpu.sync_copy(packed_hbm.at[chunks], packed_v)

    z16 = jnp.zeros((16,), jnp.float32)

    def zrow(r, carry):
        for c in range(_H // 16):
            rows0[r, pl.ds(c * 16, 16)] = z16
        return carry

    lax.fori_loop(0, _C, zrow, 0, unroll=False)
    for j in range(5):
        n = _C if j < 4 else _RPT - 4 * _C
        pltpu.sync_copy(rows0.at[pl.ds(0, n)],
                        acc_sh.at[pl.ds(sid * _RPT + j * _C, n)])
    plsc.subcore_barrier()

    def unpack(k, slot):
        for c in range(_C // 16):
            w = packed_v[k, pl.ds(c * 16, 16)]
            srcr[slot, pl.ds(c * 16, 16)] = w & 0xFFFF
            dstr[slot, pl.ds(c * 16, 16)] = lax.shift_right_logical(w, 16)

    def gather(slot, buf, sem):
        pltpu.async_copy(x_hbm.at[srcr.at[slot]], buf, sem)

    def gwait(buf, sem):
        pltpu.make_async_copy(x_hbm.at[srcr.at[0]], buf, sem).wait()

    def scatter(buf, slot):
        pltpu.sync_copy(buf, acc_sh.at[dstr.at[slot]], add=True)

    unpack(0, 0)
    gather(0, rows0, sem0)

    def body2(i, carry):
        k0 = 2 * i
        unpack(k0 + 1, 1)
        gather(1, rows1, sem1)
        gwait(rows0, sem0)
        scatter(rows0, 0)

        @pl.when(i < _CPT // 2 - 1)
        def _():
            unpack(k0 + 2, 0)
            gather(0, rows0, sem0)

        gwait(rows1, sem1)
        scatter(rows1, 1)
        return carry

    lax.fori_loop(0, _CPT // 2, body2, 0, unroll=False)
    plsc.subcore_barrier()

    pltpu.sync_copy(acc_sh.at[rows], agg_hbm.at[cid, rows])


_sc_agg = pl.kernel(
    _sc_agg_body,
    out_type=jax.ShapeDtypeStruct((_NC, _NROW, _H), jnp.float32),
    mesh=_MESH,
    scratch_types=(
        pltpu.VMEM((_CPT, _C), jnp.int32),
        pltpu.VMEM((2, _C), jnp.int32),
        pltpu.VMEM((2, _C), jnp.int32),
        pltpu.VMEM((_C, _H), jnp.float32),
        pltpu.VMEM((_C, _H), jnp.float32),
        pltpu.VMEM_SHARED((_NROW, _H), jnp.float32),
        pltpu.SemaphoreType.DMA,
        pltpu.SemaphoreType.DMA,
    ),
)


def _sc_deg_body(dst_hbm, zerosd_hbm, ones_hbm, deg_hbm,
                 dst_v, ones_v, deg_sh, gsem):
    cid = lax.axis_index("c")
    sid = lax.axis_index("s")
    rows = pl.ds(sid * _RPT, _RPT)
    chunks = pl.ds((cid * _NS + sid) * _CPT, _CPT)

    pltpu.sync_copy(dst_hbm.at[chunks], dst_v)
    pltpu.sync_copy(zerosd_hbm, deg_sh.at[rows])
    pltpu.sync_copy(ones_hbm, ones_v)
    plsc.subcore_barrier()

    def chunk(k, carry):
        pltpu.sync_copy(ones_v, deg_sh.at[dst_v.at[k]], add=True)
        return carry

    lax.fori_loop(0, _CPT, chunk, 0, unroll=False)
    plsc.subcore_barrier()
    pltpu.sync_copy(deg_sh.at[rows], deg_hbm.at[cid, rows])


_sc_deg = pl.kernel(
    _sc_deg_body,
    out_type=jax.ShapeDtypeStruct((_NC, _NROW, _DW), jnp.float32),
    mesh=_MESH,
    scratch_types=(
        pltpu.VMEM((_CPT, _C), jnp.int32),
        pltpu.VMEM((_C, _DW), jnp.float32),
        pltpu.VMEM_SHARED((_NROW, _DW), jnp.float32),
        pltpu.SemaphoreType.DMA,
    ),
)


def _tc_layer_body(aggp, x_ref, degw, wl, wr, b, g, bb, out):
    agg = aggp[0, :_N, :] + aggp[1, :_N, :]
    x = x_ref[...]
    deg = jnp.maximum(degw[0, :_N, 0:1] + degw[1, :_N, 0:1], 1.0)
    y = (jnp.dot(agg / deg, wl[...], preferred_element_type=jnp.float32)
         + jnp.dot(x, wr[...], preferred_element_type=jnp.float32)
         + b[...])
    mu = jnp.mean(y, axis=0, keepdims=True)
    var = jnp.mean((y - mu) ** 2, axis=0, keepdims=True)
    y = (y - mu) * lax.rsqrt(var + _EPS) * g[...] + bb[...]
    out[...] = jnp.maximum(y, 0.0)


_tc_layer = pl.pallas_call(
    _tc_layer_body,
    out_shape=jax.ShapeDtypeStruct((_N, _H), jnp.float32),
)


def _tc_head1_body(x2d, w1, b1, w2, b2, out):
    h = jnp.dot(x2d[...], w1[...], preferred_element_type=jnp.float32)
    h = jnp.maximum(h + b1[...], 0.0)
    out[...] = jnp.dot(h, w2[...], preferred_element_type=jnp.float32) + b2[...]


_tc_head1 = pl.pallas_call(
    _tc_head1_body,
    out_shape=jax.ShapeDtypeStruct((_N * _H // _MNN, 1), jnp.float32),
)


def _tc_head2_body(t, g, bb, w1, b1, w2, b2, out):
    x = t[...]
    mu = jnp.mean(x, axis=0, keepdims=True)
    var = jnp.mean((x - mu) ** 2, axis=0, keepdims=True)
    x = (x - mu) * lax.rsqrt(var + _EPS) * g[...] + bb[...]
    x = jnp.maximum(x, 0.0)
    h = jnp.maximum(
        jnp.dot(x, w1[...], preferred_element_type=jnp.float32) + b1[...], 0.0)
    out[...] = (jnp.dot(h, w2[...], preferred_element_type=jnp.float32)
                + b2[...])


_tc_head2 = pl.pallas_call(
    _tc_head2_body,
    out_shape=jax.ShapeDtypeStruct((_N * _H // _MNN // _H, _OUT), jnp.float32),
)


def kernel(x_ori, gamora0, gamora1, gamora2, edge_index, Wl, bl, Wr, bn_g,
           bn_b, m1_W1, m1_b1, m1_W2, m1_b2, bn2_g, bn2_b, m2_W1, m2_b1,
           m2_W2, m2_b2):
    x = jnp.concatenate([x_ori, gamora0, gamora1, gamora2], axis=1)

    pad = _EPAD - _E
    src = jnp.concatenate(
        [edge_index[0], jnp.zeros((pad,), jnp.int32)]).reshape(_NCHUNK, _C)
    dst = jnp.concatenate(
        [edge_index[1], jnp.full((pad,), _N, jnp.int32)]).reshape(_NCHUNK, _C)

    packed = jnp.bitwise_or(src, jnp.left_shift(dst, 16))
    zerosd = jnp.zeros((_RPT, _DW), jnp.float32)
    ones = jnp.ones((_C, _DW), jnp.float32)

    degw = _sc_deg(dst, zerosd, ones)
    for i in range(_L):
        aggp = _sc_agg(x, packed)
        x = _tc_layer(aggp, x, degw, Wl[i], Wr[i], bl[i].reshape(1, _H),
                      bn_g[i].reshape(1, _H), bn_b[i].reshape(1, _H))

    x2d = x.reshape(_N * _H // _MNN, _MNN)
    t = _tc_head1(x2d, m1_W1, m1_b1.reshape(1, _H), m1_W2,
                  m1_b2.reshape(1, 1))
    t10 = t.reshape(_N * _H // _MNN // _H, _H)
    out = _tc_head2(t10, bn2_g.reshape(1, _H), bn2_b.reshape(1, _H),
                    m2_W1, m2_b1.reshape(1, _H), m2_W2,
                    m2_b2.reshape(1, _OUT))
    return out


# speedup vs baseline: 10.4242x; 3.3752x over previous
import functools

import jax
import jax.numpy as jnp
from jax import lax
from jax.experimental import pallas as pl
from jax.experimental.pallas import tpu as pltpu
from jax.experimental.pallas import tpu_sc as plsc

_N = 10000
_E = 320000
_H = 128
_OUT = 2
_MNN = 1000
_L = 4
_EPS = 1e-5

_C = 128
_EPAD = 327680
_NCHUNK = _EPAD // _C
_NS = 16
_NC = 2
_CPT = _NCHUNK // (_NC * _NS)
_RPT = 632
_NROW = _NS * _RPT
_DW = 128


_MESH = plsc.VectorSubcoreMesh(core_axis_name="c", subcore_axis_name="s")


def _sc_agg_body(x_hbm, packed_hbm, agg_hbm,
                 packed_v, srcr, dstr, rows0, rows1, acc_sh, sem0, sem1):
    cid = lax.axis_index("c")
    sid = lax.axis_index("s")
    rows = pl.ds(sid * _RPT, _RPT)
    chunks = pl.ds((cid * _NS + sid) * _CPT, _CPT)

    pltpu.sync_copy(packed_hbm.at[chunks], packed_v)

    z16 = jnp.zeros((16,), jnp.float32)

    def zrow(r, carry):
        for c in range(_H // 16):
            rows0[r, pl.ds(c * 16, 16)] = z16
        return carry

    lax.fori_loop(0, _C, zrow, 0, unroll=False)
    for j in range(5):
        n = _C if j < 4 else _RPT - 4 * _C
        pltpu.sync_copy(rows0.at[pl.ds(0, n)],
                        acc_sh.at[pl.ds(sid * _RPT + j * _C, n)])
    plsc.subcore_barrier()

    def unpack(k, slot):
        for c in range(_C // 16):
            w = packed_v[k, pl.ds(c * 16, 16)]
            srcr[slot, pl.ds(c * 16, 16)] = w & 0xFFFF
            dstr[slot, pl.ds(c * 16, 16)] = lax.shift_right_logical(w, 16)

    def gather(slot, buf, sem):
        pltpu.async_copy(x_hbm.at[srcr.at[slot]], buf, sem)

    def gwait(buf, sem):
        pltpu.make_async_copy(x_hbm.at[srcr.at[0]], buf, sem).wait()

    def scatter(buf, slot):
        pltpu.sync_copy(buf, acc_sh.at[dstr.at[slot]], add=True)

    unpack(0, 0)
    gather(0, rows0, sem0)

    def body2(i, carry):
        k0 = 2 * i
        unpack(k0 + 1, 1)
        gather(1, rows1, sem1)
        gwait(rows0, sem0)
        scatter(rows0, 0)

        @pl.when(i < _CPT // 2 - 1)
        def _():
            unpack(k0 + 2, 0)
            gather(0, rows0, sem0)

        gwait(rows1, sem1)
        scatter(rows1, 1)
        return carry

    lax.fori_loop(0, _CPT // 2, body2, 0, unroll=False)
    plsc.subcore_barrier()

    pltpu.sync_copy(acc_sh.at[rows], agg_hbm.at[cid, rows])


_sc_agg = pl.kernel(
    _sc_agg_body,
    out_type=jax.ShapeDtypeStruct((_NC, _NROW, _H), jnp.float32),
    mesh=_MESH,
    scratch_types=(
        pltpu.VMEM((_CPT, _C), jnp.int32),
        pltpu.VMEM((2, _C), jnp.int32),
        pltpu.VMEM((2, _C), jnp.int32),
        pltpu.VMEM((_C, _H), jnp.float32),
        pltpu.VMEM((_C, _H), jnp.float32),
        pltpu.VMEM_SHARED((_NROW, _H), jnp.float32),
        pltpu.SemaphoreType.DMA,
        pltpu.SemaphoreType.DMA,
    ),
)


def _sc_deg_body(dst_hbm, zerosd_hbm, ones_hbm, deg_hbm,
                 dst_v, ones_v, deg_sh, gsem):
    cid = lax.axis_index("c")
    sid = lax.axis_index("s")
    rows = pl.ds(sid * _RPT, _RPT)
    chunks = pl.ds((cid * _NS + sid) * _CPT, _CPT)

    pltpu.sync_copy(dst_hbm.at[chunks], dst_v)
    pltpu.sync_copy(zerosd_hbm, deg_sh.at[rows])
    pltpu.sync_copy(ones_hbm, ones_v)
    plsc.subcore_barrier()

    def chunk(k, carry):
        pltpu.sync_copy(ones_v, deg_sh.at[dst_v.at[k]], add=True)
        return carry

    lax.fori_loop(0, _CPT, chunk, 0, unroll=False)
    plsc.subcore_barrier()
    pltpu.sync_copy(deg_sh.at[rows], deg_hbm.at[cid, rows])


_sc_deg = pl.kernel(
    _sc_deg_body,
    out_type=jax.ShapeDtypeStruct((_NC, _NROW, _DW), jnp.float32),
    mesh=_MESH,
    scratch_types=(
        pltpu.VMEM((_CPT, _C), jnp.int32),
        pltpu.VMEM((_C, _DW), jnp.float32),
        pltpu.VMEM_SHARED((_NROW, _DW), jnp.float32),
        pltpu.SemaphoreType.DMA,
    ),
)


def _tc_layer_body(aggp, x_ref, degw, wl, wr, b, g, bb, out):
    agg = aggp[0, :_N, :] + aggp[1, :_N, :]
    x = x_ref[...]
    deg = jnp.maximum(degw[0, :_N, 0:1] + degw[1, :_N, 0:1], 1.0)
    y = (jnp.dot(agg / deg, wl[...], preferred_element_type=jnp.float32)
         + jnp.dot(x, wr[...], preferred_element_type=jnp.float32)
         + b[...])
    mu = jnp.mean(y, axis=0, keepdims=True)
    var = jnp.mean((y - mu) ** 2, axis=0, keepdims=True)
    y = (y - mu) * lax.rsqrt(var + _EPS) * g[...] + bb[...]
    out[...] = jnp.maximum(y, 0.0)


_tc_layer = pl.pallas_call(
    _tc_layer_body,
    out_shape=jax.ShapeDtypeStruct((_N, _H), jnp.float32),
)


def _tc_head1_body(x2d, w1, b1, w2, b2, out):
    h = jnp.dot(x2d[...], w1[...], preferred_element_type=jnp.float32)
    h = jnp.maximum(h + b1[...], 0.0)
    out[...] = jnp.dot(h, w2[...], preferred_element_type=jnp.float32) + b2[...]


_tc_head1 = pl.pallas_call(
    _tc_head1_body,
    out_shape=jax.ShapeDtypeStruct((_N * _H // _MNN, 1), jnp.float32),
)


def _tc_head2_body(t, g, bb, w1, b1, w2, b2, out):
    x = t[...]
    mu = jnp.mean(x, axis=0, keepdims=True)
    var = jnp.mean((x - mu) ** 2, axis=0, keepdims=True)
    x = (x - mu) * lax.rsqrt(var + _EPS) * g[...] + bb[...]
    x = jnp.maximum(x, 0.0)
    h = jnp.maximum(
        jnp.dot(x, w1[...], preferred_element_type=jnp.float32) + b1[...], 0.0)
    out[...] = (jnp.dot(h, w2[...], preferred_element_type=jnp.float32)
                + b2[...])


_tc_head2 = pl.pallas_call(
    _tc_head2_body,
    out_shape=jax.ShapeDtypeStruct((_N * _H // _MNN // _H, _OUT), jnp.float32),
)


def kernel(x_ori, gamora0, gamora1, gamora2, edge_index, Wl, bl, Wr, bn_g,
           bn_b, m1_W1, m1_b1, m1_W2, m1_b2, bn2_g, bn2_b, m2_W1, m2_b1,
           m2_W2, m2_b2):
    x = jnp.concatenate([x_ori, gamora0, gamora1, gamora2], axis=1)

    pad = _EPAD - _E
    pad_src = jnp.arange(pad, dtype=jnp.int32) % _N
    pad_dst = jnp.arange(pad, dtype=jnp.int32) % (_NROW - _N) + _N
    src = jnp.concatenate([edge_index[0], pad_src]).reshape(_NCHUNK, _C)
    dst = jnp.concatenate([edge_index[1], pad_dst]).reshape(_NCHUNK, _C)

    packed = jnp.bitwise_or(src, jnp.left_shift(dst, 16))
    zerosd = jnp.zeros((_RPT, _DW), jnp.float32)
    ones = jnp.ones((_C, _DW), jnp.float32)

    degw = _sc_deg(dst, zerosd, ones)
    for i in range(_L):
        aggp = _sc_agg(x, packed)
        x = _tc_layer(aggp, x, degw, Wl[i], Wr[i], bl[i].reshape(1, _H),
                      bn_g[i].reshape(1, _H), bn_b[i].reshape(1, _H))

    x2d = x.reshape(_N * _H // _MNN, _MNN)
    t = _tc_head1(x2d, m1_W1, m1_b1.reshape(1, _H), m1_W2,
                  m1_b2.reshape(1, 1))
    t10 = t.reshape(_N * _H // _MNN // _H, _H)
    out = _tc_head2(t10, bn2_g.reshape(1, _H), bn2_b.reshape(1, _H),
                    m2_W1, m2_b1.reshape(1, _H), m2_W2,
                    m2_b2.reshape(1, _OUT))
    return out


# speedup vs baseline: 11.7268x; 1.1250x over previous
import functools

import jax
import jax.numpy as jnp
from jax import lax
from jax.experimental import pallas as pl
from jax.experimental.pallas import tpu as pltpu
from jax.experimental.pallas import tpu_sc as plsc

_N = 10000
_E = 320000
_H = 128
_OUT = 2
_MNN = 1000
_L = 4
_EPS = 1e-5

_C = 128
_EPAD = 327680
_NCHUNK = _EPAD // _C
_NS = 16
_NC = 2
_CPT = _NCHUNK // (_NC * _NS)
_RPT = 632
_NROW = _NS * _RPT
_DW = 16


_MESH = plsc.VectorSubcoreMesh(core_axis_name="c", subcore_axis_name="s")


def _sc_agg_body(x_hbm, packed_hbm, agg_hbm,
                 packed_v, srcr, dstr, rows0, rows1, acc_sh, sem0, sem1):
    cid = lax.axis_index("c")
    sid = lax.axis_index("s")
    rows = pl.ds(sid * _RPT, _RPT)
    chunks = pl.ds((cid * _NS + sid) * _CPT, _CPT)

    pltpu.sync_copy(packed_hbm.at[chunks], packed_v)

    z16 = jnp.zeros((16,), jnp.float32)

    def zrow(r, carry):
        for c in range(_H // 16):
            rows0[r, pl.ds(c * 16, 16)] = z16
        return carry

    lax.fori_loop(0, _C, zrow, 0, unroll=False)
    for j in range(5):
        n = _C if j < 4 else _RPT - 4 * _C
        pltpu.sync_copy(rows0.at[pl.ds(0, n)],
                        acc_sh.at[pl.ds(sid * _RPT + j * _C, n)])
    plsc.subcore_barrier()

    def unpack(k, slot):
        for c in range(_C // 16):
            w = packed_v[k, pl.ds(c * 16, 16)]
            srcr[slot, pl.ds(c * 16, 16)] = w & 0xFFFF
            dstr[slot, pl.ds(c * 16, 16)] = lax.shift_right_logical(w, 16)

    def gather(slot, buf, sem):
        pltpu.async_copy(x_hbm.at[srcr.at[slot]], buf, sem)

    def gwait(buf, sem):
        pltpu.make_async_copy(x_hbm.at[srcr.at[0]], buf, sem).wait()

    def scatter(buf, slot):
        pltpu.sync_copy(buf, acc_sh.at[dstr.at[slot]], add=True)

    unpack(0, 0)
    gather(0, rows0, sem0)

    def body2(i, carry):
        k0 = 2 * i
        unpack(k0 + 1, 1)
        gather(1, rows1, sem1)
        gwait(rows0, sem0)
        scatter(rows0, 0)

        @pl.when(i < _CPT // 2 - 1)
        def _():
            unpack(k0 + 2, 0)
            gather(0, rows0, sem0)

        gwait(rows1, sem1)
        scatter(rows1, 1)
        return carry

    lax.fori_loop(0, _CPT // 2, body2, 0, unroll=False)
    plsc.subcore_barrier()

    pltpu.sync_copy(acc_sh.at[rows], agg_hbm.at[cid, rows])


_sc_agg = pl.kernel(
    _sc_agg_body,
    out_type=jax.ShapeDtypeStruct((_NC, _NROW, _H), jnp.float32),
    mesh=_MESH,
    scratch_types=(
        pltpu.VMEM((_CPT, _C), jnp.int32),
        pltpu.VMEM((2, _C), jnp.int32),
        pltpu.VMEM((2, _C), jnp.int32),
        pltpu.VMEM((_C, _H), jnp.float32),
        pltpu.VMEM((_C, _H), jnp.float32),
        pltpu.VMEM_SHARED((_NROW, _H), jnp.float32),
        pltpu.SemaphoreType.DMA,
        pltpu.SemaphoreType.DMA,
    ),
)


def _sc_deg_body(dst_hbm, zerosd_hbm, ones_hbm, deg_hbm,
                 dst_v, ones_v, deg_sh, gsem):
    cid = lax.axis_index("c")
    sid = lax.axis_index("s")
    rows = pl.ds(sid * _RPT, _RPT)
    chunks = pl.ds((cid * _NS + sid) * _CPT, _CPT)

    pltpu.sync_copy(dst_hbm.at[chunks], dst_v)
    pltpu.sync_copy(zerosd_hbm, deg_sh.at[rows])
    pltpu.sync_copy(ones_hbm, ones_v)
    plsc.subcore_barrier()

    def chunk(k, carry):
        pltpu.sync_copy(ones_v, deg_sh.at[dst_v.at[k]], add=True)
        return carry

    lax.fori_loop(0, _CPT, chunk, 0, unroll=False)
    plsc.subcore_barrier()
    pltpu.sync_copy(deg_sh.at[rows], deg_hbm.at[cid, rows])


_sc_deg = pl.kernel(
    _sc_deg_body,
    out_type=jax.ShapeDtypeStruct((_NC, _NROW, _DW), jnp.float32),
    mesh=_MESH,
    compiler_params=pltpu.CompilerParams(use_tc_tiling_on_sc=False),
    scratch_types=(
        pltpu.VMEM((_CPT, _C), jnp.int32),
        pltpu.VMEM((_C, _DW), jnp.float32),
        pltpu.VMEM_SHARED((_NROW, _DW), jnp.float32),
        pltpu.SemaphoreType.DMA,
    ),
)


def _tc_layer_body(aggp, x_ref, degn, wl, wr, b, g, bb, out):
    agg = aggp[0, :_N, :] + aggp[1, :_N, :]
    x = x_ref[...]
    deg = jnp.maximum(degn[0] + degn[1], 1.0)
    y = (jnp.dot(agg / deg, wl[...], preferred_element_type=jnp.float32)
         + jnp.dot(x, wr[...], preferred_element_type=jnp.float32)
         + b[...])
    mu = jnp.mean(y, axis=0, keepdims=True)
    var = jnp.mean((y - mu) ** 2, axis=0, keepdims=True)
    y = (y - mu) * lax.rsqrt(var + _EPS) * g[...] + bb[...]
    out[...] = jnp.maximum(y, 0.0)


_tc_layer = pl.pallas_call(
    _tc_layer_body,
    out_shape=jax.ShapeDtypeStruct((_N, _H), jnp.float32),
)


def _tc_head1_body(x2d, w1, b1, w2, b2, out):
    h = jnp.dot(x2d[...], w1[...], preferred_element_type=jnp.float32)
    h = jnp.maximum(h + b1[...], 0.0)
    out[...] = jnp.dot(h, w2[...], preferred_element_type=jnp.float32) + b2[...]


_tc_head1 = pl.pallas_call(
    _tc_head1_body,
    out_shape=jax.ShapeDtypeStruct((_N * _H // _MNN, 1), jnp.float32),
)


def _tc_head2_body(t, g, bb, w1, b1, w2, b2, out):
    x = t[...]
    mu = jnp.mean(x, axis=0, keepdims=True)
    var = jnp.mean((x - mu) ** 2, axis=0, keepdims=True)
    x = (x - mu) * lax.rsqrt(var + _EPS) * g[...] + bb[...]
    x = jnp.maximum(x, 0.0)
    h = jnp.maximum(
        jnp.dot(x, w1[...], preferred_element_type=jnp.float32) + b1[...], 0.0)
    out[...] = (jnp.dot(h, w2[...], preferred_element_type=jnp.float32)
                + b2[...])


_tc_head2 = pl.pallas_call(
    _tc_head2_body,
    out_shape=jax.ShapeDtypeStruct((_N * _H // _MNN // _H, _OUT), jnp.float32),
)


def kernel(x_ori, gamora0, gamora1, gamora2, edge_index, Wl, bl, Wr, bn_g,
           bn_b, m1_W1, m1_b1, m1_W2, m1_b2, bn2_g, bn2_b, m2_W1, m2_b1,
           m2_W2, m2_b2):
    x = jnp.concatenate([x_ori, gamora0, gamora1, gamora2], axis=1)

    pad = _EPAD - _E
    pad_src = jnp.arange(pad, dtype=jnp.int32) % _N
    pad_dst = jnp.arange(pad, dtype=jnp.int32) % (_NROW - _N) + _N
    src = jnp.concatenate([edge_index[0], pad_src]).reshape(_NCHUNK, _C)
    dst = jnp.concatenate([edge_index[1], pad_dst]).reshape(_NCHUNK, _C)

    packed = jnp.bitwise_or(src, jnp.left_shift(dst, 16))
    zerosd = jnp.zeros((_RPT, _DW), jnp.float32)
    ones = jnp.ones((_C, _DW), jnp.float32)

    degw = _sc_deg(dst, zerosd, ones)
    degn = degw[:, :_N, 0:1]
    for i in range(_L):
        aggp = _sc_agg(x, packed)
        x = _tc_layer(aggp, x, degn, Wl[i], Wr[i], bl[i].reshape(1, _H),
                      bn_g[i].reshape(1, _H), bn_b[i].reshape(1, _H))

    x2d = x.reshape(_N * _H // _MNN, _MNN)
    t = _tc_head1(x2d, m1_W1, m1_b1.reshape(1, _H), m1_W2,
                  m1_b2.reshape(1, 1))
    t10 = t.reshape(_N * _H // _MNN // _H, _H)
    out = _tc_head2(t10, bn2_g.reshape(1, _H), bn2_b.reshape(1, _H),
                    m2_W1, m2_b1.reshape(1, _H), m2_W2,
                    m2_b2.reshape(1, _OUT))
    return out
